# baseline (speedup 1.0000x reference)
import functools

import jax
import jax.numpy as jnp
from jax import lax
from jax.experimental import pallas as pl
from jax.experimental.pallas import tpu as pltpu
from jax.experimental.pallas import tpu_sc as plsc

N_ATOMS_ = 10000
N_BONDS_ = 320000
ATOM_FDIM_ = 128
BOND_FDIM_ = 144
H = 128
MAX_NB_ = 32
N_MOLS_ = 500

NC, NS = 2, 16
GW = 128

A_PAD = 10240
APC = A_PAD // NC
APS = APC // NS
AWIN = APS * MAX_NB_ // GW

B_PAD = 327680
WPW = (B_PAD // GW) // (NC * NS)

_mesh = plsc.VectorSubcoreMesh(core_axis_name="c", subcore_axis_name="s")


_K = 4
_RCH = 64
_NR = APS // _RCH
_RWIN = _RCH * MAX_NB_ // GW
_NBAT = _RWIN // _K


def _gsum_body(msg_hbm, a2b_hbm, sidx_hbm, zeros_hbm, out_hbm,
               gidx_v, sidx_v, gbufs_v, accum_sh, gsem):
    c = lax.axis_index("c")
    s = lax.axis_index("s")
    atom_base = pl.multiple_of(c * APC + s * APS, APS)
    flat_base = pl.multiple_of(atom_base * MAX_NB_, APS * MAX_NB_)
    row_base = pl.multiple_of(atom_base // 4, AWIN)
    stripe = pl.multiple_of(s * _RCH, _RCH)
    pltpu.sync_copy(a2b_hbm.at[pl.ds(flat_base, AWIN * GW)], gidx_v)
    pltpu.sync_copy(sidx_hbm.at[pl.ds(row_base, AWIN)], sidx_v)

    def gidx(w):
        return gidx_v.at[pl.ds(w * GW, GW)]

    @pl.loop(0, _NR)
    def _(r):
        pltpu.sync_copy(zeros_hbm, accum_sh.at[pl.ds(stripe, _RCH)])

        @pl.loop(0, _NBAT)
        def _(b):
            w0 = r * _RWIN + b * _K
            for t in range(_K):
                pltpu.async_copy(msg_hbm.at[gidx(w0 + t)], gbufs_v.at[t],
                                 gsem)
            for t in range(_K):
                pltpu.make_async_copy(msg_hbm.at[gidx(w0 + t)],
                                      gbufs_v.at[t], gsem).wait()
            for t in range(_K):
                pltpu.sync_copy(gbufs_v.at[t],
                                accum_sh.at[sidx_v.at[w0 + t, 0]], add=True)

        pltpu.sync_copy(accum_sh.at[pl.ds(stripe, _RCH)],
                        out_hbm.at[pl.ds(atom_base + r * _RCH, _RCH)])


@jax.jit
def _sc_gather_sum(message, a2b_flat, sidx_rows, zeros_tile):
    k = pl.kernel(
        _gsum_body,
        out_type=jax.ShapeDtypeStruct((A_PAD, H), jnp.float32),
        mesh=_mesh,
        scratch_types=[
            pltpu.VMEM((AWIN * GW,), jnp.int32),
            pltpu.VMEM((AWIN, 1, GW), jnp.int32),
            pltpu.VMEM((_K, GW, H), jnp.float32),
            pltpu.VMEM_SHARED((NS * _RCH, H), jnp.float32),
            pltpu.SemaphoreType.DMA,
        ],
    )
    return k(message, a2b_flat, sidx_rows, zeros_tile)


_BK = 2


def _edge_body(msg_hbm, amsg_hbm, brev_hbm, b2a_hbm, out_hbm,
               i1_v, i2_v, b1s_v, b2s_v, gsem):
    c = lax.axis_index("c")
    s = lax.axis_index("s")
    wid = c * NS + s
    base = pl.multiple_of(wid * WPW * GW, WPW * GW)
    pltpu.sync_copy(brev_hbm.at[pl.ds(base, WPW * GW)], i1_v)
    pltpu.sync_copy(b2a_hbm.at[pl.ds(base, WPW * GW)], i2_v)

    def sub(k):
        @pl.loop(0, GW, step=4)
        def _(r):
            for dr in range(4):
                for ch in range(H // 16):
                    sl = pl.ds(ch * 16, 16)
                    b2s_v[k, r + dr, sl] = b2s_v[k, r + dr, sl] - b1s_v[k, r + dr, sl]

    @pl.loop(0, WPW // _BK)
    def _(bt):
        w0 = bt * _BK
        for t in range(_BK):
            w = w0 + t
            pltpu.async_copy(msg_hbm.at[i1_v.at[pl.ds(w * GW, GW)]],
                             b1s_v.at[t], gsem)
            pltpu.async_copy(amsg_hbm.at[i2_v.at[pl.ds(w * GW, GW)]],
                             b2s_v.at[t], gsem)
        for t in range(_BK):
            w = w0 + t
            pltpu.make_async_copy(msg_hbm.at[i1_v.at[pl.ds(w * GW, GW)]],
                                  b1s_v.at[t], gsem).wait()
            pltpu.make_async_copy(amsg_hbm.at[i2_v.at[pl.ds(w * GW, GW)]],
                                  b2s_v.at[t], gsem).wait()
        for t in range(_BK):
            sub(t)
            pltpu.sync_copy(b2s_v.at[t],
                            out_hbm.at[pl.ds(base + (w0 + t) * GW, GW)])


@jax.jit
def _sc_edge(message, a_msg, brev_rows, b2a_rows):
    k = pl.kernel(
        _edge_body,
        out_type=jax.ShapeDtypeStruct((B_PAD, H), jnp.float32),
        mesh=_mesh,
        scratch_types=[
            pltpu.VMEM((WPW * GW,), jnp.int32),
            pltpu.VMEM((WPW * GW,), jnp.int32),
            pltpu.VMEM((_BK, GW, H), jnp.float32),
            pltpu.VMEM((_BK, GW, H), jnp.float32),
            pltpu.SemaphoreType.DMA,
        ],
    )
    return k(message, a_msg, brev_rows, b2a_rows)


_TB = 3200
_NBLK = N_BONDS_ // _TB


def _init_body(fb_ref, wi_ref, inp_ref, msg_ref):
    x = jnp.dot(fb_ref[...], wi_ref[...], preferred_element_type=jnp.float32)
    inp_ref[...] = x
    msg_ref[...] = jnp.maximum(x, 0.0)


@jax.jit
def _tc_init(f_bonds, W_i):
    return pl.pallas_call(
        _init_body,
        grid=(_NBLK,),
        in_specs=[
            pl.BlockSpec((_TB, BOND_FDIM_), lambda i: (i, 0)),
            pl.BlockSpec((BOND_FDIM_, H), lambda i: (0, 0)),
        ],
        out_specs=[
            pl.BlockSpec((_TB, H), lambda i: (i, 0)),
            pl.BlockSpec((_TB, H), lambda i: (i, 0)),
        ],
        out_shape=[
            jax.ShapeDtypeStruct((N_BONDS_, H), jnp.float32),
            jax.ShapeDtypeStruct((N_BONDS_, H), jnp.float32),
        ],
    )(f_bonds, W_i)


def _update_body(inp_ref, pre_ref, wh_ref, out_ref):
    x = jnp.dot(pre_ref[...], wh_ref[...], preferred_element_type=jnp.float32)
    out_ref[...] = jnp.maximum(inp_ref[...] + x, 0.0)


@jax.jit
def _tc_update(inp, pre, W_h):
    return pl.pallas_call(
        _update_body,
        grid=(_NBLK,),
        in_specs=[
            pl.BlockSpec((_TB, H), lambda i: (i, 0)),
            pl.BlockSpec((_TB, H), lambda i: (i, 0)),
            pl.BlockSpec((H, H), lambda i: (0, 0)),
        ],
        out_specs=pl.BlockSpec((_TB, H), lambda i: (i, 0)),
        out_shape=jax.ShapeDtypeStruct((N_BONDS_, H), jnp.float32),
    )(inp, pre, W_h)


_RB = 1024
_RN = A_PAD // _RB
_SEG = 512


def _readout_body(fa_ref, am_ref, mid_ref, woa_ref, woh_ref, bo_ref,
                  out_ref, sums_scr, cnts_scr):
    i = pl.program_id(0)
    hid = (
        jnp.dot(fa_ref[...], woa_ref[...], preferred_element_type=jnp.float32)
        + jnp.dot(am_ref[...], woh_ref[...], preferred_element_type=jnp.float32)
        + bo_ref[...]
    )
    ids = mid_ref[0]
    seg_iota = lax.broadcasted_iota(jnp.int32, (_SEG, _RB), 0)
    onehot_t = (ids == seg_iota).astype(jnp.float32)
    contrib = jnp.dot(onehot_t, hid, preferred_element_type=jnp.float32)
    cnts = jnp.dot(onehot_t, jnp.ones((_RB, H), jnp.float32),
                   preferred_element_type=jnp.float32)

    @pl.when(i == 0)
    def _():
        sums_scr[...] = jnp.zeros_like(sums_scr)
        cnts_scr[...] = jnp.zeros_like(cnts_scr)

    sums_scr[...] += contrib
    cnts_scr[...] += cnts

    @pl.when(i == _RN - 1)
    def _():
        out_ref[...] = sums_scr[...] / jnp.maximum(cnts_scr[...], 1.0)


@jax.jit
def _tc_readout(fa_p, a_msg, mid_r, Wo_a, Wo_h, bo_r):
    return pl.pallas_call(
        _readout_body,
        grid=(_RN,),
        in_specs=[
            pl.BlockSpec((_RB, ATOM_FDIM_), lambda i: (i, 0)),
            pl.BlockSpec((_RB, H), lambda i: (i, 0)),
            pl.BlockSpec((1, 1, _RB), lambda i: (i, 0, 0)),
            pl.BlockSpec((ATOM_FDIM_, H), lambda i: (0, 0)),
            pl.BlockSpec((H, H), lambda i: (0, 0)),
            pl.BlockSpec((1, H), lambda i: (0, 0)),
        ],
        out_specs=pl.BlockSpec((_SEG, H), lambda i: (0, 0)),
        out_shape=jax.ShapeDtypeStruct((_SEG, H), jnp.float32),
        scratch_shapes=[
            pltpu.VMEM((_SEG, H), jnp.float32),
            pltpu.VMEM((_SEG, H), jnp.float32),
        ],
    )(fa_p, a_msg, mid_r, Wo_a, Wo_h, bo_r)


def kernel(f_atoms, f_bonds, a2b, b2a, b2revb, mol_ids, W_i, W_h, W_o, b_o):
    a2b_flat = jnp.pad(a2b, ((0, A_PAD - N_ATOMS_), (0, 0))).reshape(-1)
    _al = jnp.arange(A_PAD, dtype=jnp.int32)
    sidx_rows = jnp.repeat(
        ((_al % APC) // APS) * _RCH + (_al % _RCH),
        MAX_NB_).reshape(-1, 1, GW)
    zeros_tile = jnp.zeros((_RCH, H), jnp.float32)
    b2a_p = jnp.pad(b2a, (0, B_PAD - N_BONDS_))
    brev_p = jnp.pad(b2revb, (0, B_PAD - N_BONDS_))
    fa_p = jnp.pad(f_atoms, ((0, A_PAD - N_ATOMS_), (0, 0)))
    mid_r = jnp.pad(mol_ids, (0, A_PAD - N_ATOMS_),
                    constant_values=N_MOLS_).reshape(_RN, 1, _RB)
    Wo_a = W_o[:ATOM_FDIM_]
    Wo_h = W_o[ATOM_FDIM_:]
    bo_r = b_o.reshape(1, H)

    inp, message = _tc_init(f_bonds, W_i)
    for _ in range(2):
        a_msg = _sc_gather_sum(message, a2b_flat, sidx_rows, zeros_tile)
        pre = _sc_edge(message, a_msg, brev_p, b2a_p)
        message = _tc_update(inp, pre, W_h)
    a_msg = _sc_gather_sum(message, a2b_flat, sidx_rows, zeros_tile)
    out = _tc_readout(fa_p, a_msg, mid_r, Wo_a, Wo_h, bo_r)
    return out[:N_MOLS_]


# speedup vs baseline: 1.1144x; 1.1144x over previous
import functools

import jax
import jax.numpy as jnp
from jax import lax
from jax.experimental import pallas as pl
from jax.experimental.pallas import tpu as pltpu
from jax.experimental.pallas import tpu_sc as plsc

N_ATOMS_ = 10000
N_BONDS_ = 320000
ATOM_FDIM_ = 128
BOND_FDIM_ = 144
H = 128
MAX_NB_ = 32
N_MOLS_ = 500

NC, NS = 2, 16
GW = 128

A_PAD = 10240
APC = A_PAD // NC
APS = APC // NS
AWIN = APS * MAX_NB_ // GW

B_PAD = 327680
WPW = (B_PAD // GW) // (NC * NS)

_mesh = plsc.VectorSubcoreMesh(core_axis_name="c", subcore_axis_name="s")


_K = 4
_RCH = 64
_NR = APS // _RCH
_RWIN = _RCH * MAX_NB_ // GW
_NBAT = _RWIN // _K


def _gsum_body(msg_hbm, a2b_hbm, sidx_hbm, zeros_hbm, out_hbm,
               gidx_v, sidx_v, gbufs_v, accum_sh, gsem):
    c = lax.axis_index("c")
    s = lax.axis_index("s")
    atom_base = pl.multiple_of(c * APC + s * APS, APS)
    flat_base = pl.multiple_of(atom_base * MAX_NB_, APS * MAX_NB_)
    row_base = pl.multiple_of(atom_base // 4, AWIN)
    stripe = pl.multiple_of(s * _RCH, _RCH)
    pltpu.sync_copy(a2b_hbm.at[pl.ds(flat_base, AWIN * GW)], gidx_v)
    pltpu.sync_copy(sidx_hbm.at[pl.ds(row_base, AWIN)], sidx_v)

    def gidx(w):
        return gidx_v.at[pl.ds(w * GW, GW)]

    _G = _K // 2
    _NG = _RWIN // _G

    def fire(r, g):
        p = g % 2
        for t in range(_G):
            pltpu.async_copy(msg_hbm.at[gidx(r * _RWIN + g * _G + t)],
                             gbufs_v.at[p * _G + t], gsem.at[p])

    def drain(r, g):
        p = g % 2
        for t in range(_G):
            pltpu.make_async_copy(msg_hbm.at[gidx(r * _RWIN + g * _G + t)],
                                  gbufs_v.at[p * _G + t], gsem.at[p]).wait()

    def scat(r, g):
        p = g % 2
        for t in range(_G):
            pltpu.sync_copy(gbufs_v.at[p * _G + t],
                            accum_sh.at[sidx_v.at[r * _RWIN + g * _G + t, 0]],
                            add=True)

    @pl.loop(0, _NR)
    def _(r):
        pltpu.sync_copy(zeros_hbm, accum_sh.at[pl.ds(stripe, _RCH)])
        fire(r, 0)
        for g in range(_NG):
            if g < _NG - 1:
                fire(r, g + 1)
            drain(r, g)
            scat(r, g)
        pltpu.sync_copy(accum_sh.at[pl.ds(stripe, _RCH)],
                        out_hbm.at[pl.ds(atom_base + r * _RCH, _RCH)])


@jax.jit
def _sc_gather_sum(message, a2b_flat, sidx_rows, zeros_tile):
    k = pl.kernel(
        _gsum_body,
        out_type=jax.ShapeDtypeStruct((A_PAD, H), jnp.float32),
        mesh=_mesh,
        scratch_types=[
            pltpu.VMEM((AWIN * GW,), jnp.int32),
            pltpu.VMEM((AWIN, 1, GW), jnp.int32),
            pltpu.VMEM((_K, GW, H), jnp.float32),
            pltpu.VMEM_SHARED((NS * _RCH, H), jnp.float32),
            pltpu.SemaphoreType.DMA((2,)),
        ],
    )
    return k(message, a2b_flat, sidx_rows, zeros_tile)


_BK = 2


def _edge_body(msg_hbm, amsg_hbm, brev_hbm, b2a_hbm, out_hbm,
               i1_v, i2_v, b1s_v, b2s_v, gsem):
    c = lax.axis_index("c")
    s = lax.axis_index("s")
    wid = c * NS + s
    base = pl.multiple_of(wid * WPW * GW, WPW * GW)
    pltpu.sync_copy(brev_hbm.at[pl.ds(base, WPW * GW)], i1_v)
    pltpu.sync_copy(b2a_hbm.at[pl.ds(base, WPW * GW)], i2_v)

    def sub(k):
        @pl.loop(0, GW, step=4)
        def _(r):
            for dr in range(4):
                for ch in range(H // 16):
                    sl = pl.ds(ch * 16, 16)
                    b2s_v[k, r + dr, sl] = b2s_v[k, r + dr, sl] - b1s_v[k, r + dr, sl]

    def fire(w, p):
        pltpu.async_copy(msg_hbm.at[i1_v.at[pl.ds(w * GW, GW)]],
                         b1s_v.at[p], gsem.at[p])
        pltpu.async_copy(amsg_hbm.at[i2_v.at[pl.ds(w * GW, GW)]],
                         b2s_v.at[p], gsem.at[p])

    def drain(w, p):
        pltpu.make_async_copy(msg_hbm.at[i1_v.at[pl.ds(w * GW, GW)]],
                              b1s_v.at[p], gsem.at[p]).wait()
        pltpu.make_async_copy(amsg_hbm.at[i2_v.at[pl.ds(w * GW, GW)]],
                              b2s_v.at[p], gsem.at[p]).wait()

    fire(0, 0)

    @pl.loop(0, WPW // _BK)
    def _(bt):
        w0 = bt * _BK
        fire(w0 + 1, 1)
        drain(w0, 0)
        sub(0)
        pltpu.sync_copy(b2s_v.at[0], out_hbm.at[pl.ds(base + w0 * GW, GW)])

        @pl.when(bt < WPW // _BK - 1)
        def _():
            fire(w0 + 2, 0)

        drain(w0 + 1, 1)
        sub(1)
        pltpu.sync_copy(b2s_v.at[1],
                        out_hbm.at[pl.ds(base + (w0 + 1) * GW, GW)])


@jax.jit
def _sc_edge(message, a_msg, brev_rows, b2a_rows):
    k = pl.kernel(
        _edge_body,
        out_type=jax.ShapeDtypeStruct((B_PAD, H), jnp.float32),
        mesh=_mesh,
        scratch_types=[
            pltpu.VMEM((WPW * GW,), jnp.int32),
            pltpu.VMEM((WPW * GW,), jnp.int32),
            pltpu.VMEM((_BK, GW, H), jnp.float32),
            pltpu.VMEM((_BK, GW, H), jnp.float32),
            pltpu.SemaphoreType.DMA((2,)),
        ],
    )
    return k(message, a_msg, brev_rows, b2a_rows)


_TB = 3200
_NBLK = N_BONDS_ // _TB


def _init_body(fb_ref, wi_ref, inp_ref, msg_ref):
    x = jnp.dot(fb_ref[...], wi_ref[...], preferred_element_type=jnp.float32)
    inp_ref[...] = x
    msg_ref[...] = jnp.maximum(x, 0.0)


@jax.jit
def _tc_init(f_bonds, W_i):
    return pl.pallas_call(
        _init_body,
        grid=(_NBLK,),
        in_specs=[
            pl.BlockSpec((_TB, BOND_FDIM_), lambda i: (i, 0)),
            pl.BlockSpec((BOND_FDIM_, H), lambda i: (0, 0)),
        ],
        out_specs=[
            pl.BlockSpec((_TB, H), lambda i: (i, 0)),
            pl.BlockSpec((_TB, H), lambda i: (i, 0)),
        ],
        out_shape=[
            jax.ShapeDtypeStruct((N_BONDS_, H), jnp.float32),
            jax.ShapeDtypeStruct((N_BONDS_, H), jnp.float32),
        ],
    )(f_bonds, W_i)


def _update_body(inp_ref, pre_ref, wh_ref, out_ref):
    x = jnp.dot(pre_ref[...], wh_ref[...], preferred_element_type=jnp.float32)
    out_ref[...] = jnp.maximum(inp_ref[...] + x, 0.0)


@jax.jit
def _tc_update(inp, pre, W_h):
    return pl.pallas_call(
        _update_body,
        grid=(_NBLK,),
        in_specs=[
            pl.BlockSpec((_TB, H), lambda i: (i, 0)),
            pl.BlockSpec((_TB, H), lambda i: (i, 0)),
            pl.BlockSpec((H, H), lambda i: (0, 0)),
        ],
        out_specs=pl.BlockSpec((_TB, H), lambda i: (i, 0)),
        out_shape=jax.ShapeDtypeStruct((N_BONDS_, H), jnp.float32),
    )(inp, pre, W_h)


_RB = 1024
_RN = A_PAD // _RB
_SEG = 512


def _readout_body(fa_ref, am_ref, mid_ref, woa_ref, woh_ref, bo_ref,
                  out_ref, sums_scr, cnts_scr):
    i = pl.program_id(0)
    hid = (
        jnp.dot(fa_ref[...], woa_ref[...], preferred_element_type=jnp.float32)
        + jnp.dot(am_ref[...], woh_ref[...], preferred_element_type=jnp.float32)
        + bo_ref[...]
    )
    ids = mid_ref[0]
    seg_iota = lax.broadcasted_iota(jnp.int32, (_SEG, _RB), 0)
    onehot_t = (ids == seg_iota).astype(jnp.float32)
    contrib = jnp.dot(onehot_t, hid, preferred_element_type=jnp.float32)
    cnts = jnp.dot(onehot_t, jnp.ones((_RB, H), jnp.float32),
                   preferred_element_type=jnp.float32)

    @pl.when(i == 0)
    def _():
        sums_scr[...] = jnp.zeros_like(sums_scr)
        cnts_scr[...] = jnp.zeros_like(cnts_scr)

    sums_scr[...] += contrib
    cnts_scr[...] += cnts

    @pl.when(i == _RN - 1)
    def _():
        out_ref[...] = sums_scr[...] / jnp.maximum(cnts_scr[...], 1.0)


@jax.jit
def _tc_readout(fa_p, a_msg, mid_r, Wo_a, Wo_h, bo_r):
    return pl.pallas_call(
        _readout_body,
        grid=(_RN,),
        in_specs=[
            pl.BlockSpec((_RB, ATOM_FDIM_), lambda i: (i, 0)),
            pl.BlockSpec((_RB, H), lambda i: (i, 0)),
            pl.BlockSpec((1, 1, _RB), lambda i: (i, 0, 0)),
            pl.BlockSpec((ATOM_FDIM_, H), lambda i: (0, 0)),
            pl.BlockSpec((H, H), lambda i: (0, 0)),
            pl.BlockSpec((1, H), lambda i: (0, 0)),
        ],
        out_specs=pl.BlockSpec((_SEG, H), lambda i: (0, 0)),
        out_shape=jax.ShapeDtypeStruct((_SEG, H), jnp.float32),
        scratch_shapes=[
            pltpu.VMEM((_SEG, H), jnp.float32),
            pltpu.VMEM((_SEG, H), jnp.float32),
        ],
    )(fa_p, a_msg, mid_r, Wo_a, Wo_h, bo_r)


def kernel(f_atoms, f_bonds, a2b, b2a, b2revb, mol_ids, W_i, W_h, W_o, b_o):
    a2b_flat = jnp.pad(a2b, ((0, A_PAD - N_ATOMS_), (0, 0))).reshape(-1)
    _al = jnp.arange(A_PAD, dtype=jnp.int32)
    sidx_rows = jnp.repeat(
        ((_al % APC) // APS) * _RCH + (_al % _RCH),
        MAX_NB_).reshape(-1, 1, GW)
    zeros_tile = jnp.zeros((_RCH, H), jnp.float32)
    b2a_p = jnp.pad(b2a, (0, B_PAD - N_BONDS_))
    brev_p = jnp.pad(b2revb, (0, B_PAD - N_BONDS_))
    fa_p = jnp.pad(f_atoms, ((0, A_PAD - N_ATOMS_), (0, 0)))
    mid_r = jnp.pad(mol_ids, (0, A_PAD - N_ATOMS_),
                    constant_values=N_MOLS_).reshape(_RN, 1, _RB)
    Wo_a = W_o[:ATOM_FDIM_]
    Wo_h = W_o[ATOM_FDIM_:]
    bo_r = b_o.reshape(1, H)

    inp, message = _tc_init(f_bonds, W_i)
    for _ in range(2):
        a_msg = _sc_gather_sum(message, a2b_flat, sidx_rows, zeros_tile)
        pre = _sc_edge(message, a_msg, brev_p, b2a_p)
        message = _tc_update(inp, pre, W_h)
    a_msg = _sc_gather_sum(message, a2b_flat, sidx_rows, zeros_tile)
    out = _tc_readout(fa_p, a_msg, mid_r, Wo_a, Wo_h, bo_r)
    return out[:N_MOLS_]
